# baseline (speedup 1.0000x reference)
import functools

import jax
import jax.numpy as jnp
from jax import lax
from jax.experimental import pallas as pl
from jax.experimental.pallas import tpu as pltpu
from jax.experimental.pallas import tpu_sc as plsc

N_POS = 16384
N_NEG = 81920
N_PAIRS = N_POS + N_NEG
NW = 32
ROWS = N_PAIRS // 128
ROWS_PER_W = ROWS // NW
G = 4
NCH = ROWS_PER_W // G
EMB_DIM = 64


def _sc_scores(u_idx, v_idx, u_table, v_table):
    mesh = plsc.VectorSubcoreMesh(core_axis_name="c", subcore_axis_name="s")

    @functools.partial(
        pl.kernel,
        out_type=jax.ShapeDtypeStruct((ROWS, 128, 16), jnp.float32),
        mesh=mesh,
        scratch_types=[
            pltpu.VMEM((G, 128), jnp.int32),
            pltpu.VMEM((G, 128), jnp.int32),
            pltpu.VMEM((G, 128, EMB_DIM), jnp.float32),
            pltpu.VMEM((G, 128, EMB_DIM), jnp.float32),
            pltpu.VMEM((G, 128, 16), jnp.float32),
            pltpu.SemaphoreType.DMA,
        ],
        compiler_params=pltpu.CompilerParams(use_tc_tiling_on_sc=False),
    )
    def k(u_idx_hbm, v_idx_hbm, u_tab_hbm, v_tab_hbm, out_hbm,
          idx_u, idx_v, rows_u, rows_v, parts, sem):
        wid = lax.axis_index("s") * 2 + lax.axis_index("c")
        for c in range(NCH):
            r0 = wid * ROWS_PER_W + c * G
            pltpu.sync_copy(u_idx_hbm.at[pl.ds(r0, G)], idx_u)
            pltpu.sync_copy(v_idx_hbm.at[pl.ds(r0, G)], idx_v)
            cps = []
            for g in range(G):
                cps.append(pltpu.async_copy(
                    u_tab_hbm.at[idx_u.at[g]], rows_u.at[g], sem))
                cps.append(pltpu.async_copy(
                    v_tab_hbm.at[idx_v.at[g]], rows_v.at[g], sem))
            for cp in cps:
                cp.wait()
            for g in range(G):
                def body(r, _, g=g):
                    p = (rows_u[g, r, pl.ds(0, 16)] * rows_v[g, r, pl.ds(0, 16)]
                         + rows_u[g, r, pl.ds(16, 16)] * rows_v[g, r, pl.ds(16, 16)]
                         + rows_u[g, r, pl.ds(32, 16)] * rows_v[g, r, pl.ds(32, 16)]
                         + rows_u[g, r, pl.ds(48, 16)] * rows_v[g, r, pl.ds(48, 16)])
                    parts[g, r, :] = p
                    return 0
                lax.fori_loop(0, 128, body, 0, unroll=8)
            pltpu.sync_copy(parts, out_hbm.at[pl.ds(r0, G)])

    return k(u_idx, v_idx, u_table, v_table)


def _tc_loss(s):
    def body(s_ref, o_ref):
        x = s_ref[:]
        k_iota = lax.broadcasted_iota(jnp.int32, (128, 8), 0)
        j_iota = lax.broadcasted_iota(jnp.int32, (128, 8), 1)
        gm = (k_iota // 16 == j_iota).astype(jnp.float32)
        grp = lax.dot_general(x, gm, (((1,), (0,)), ((), ())),
                              preferred_element_type=jnp.float32)
        row = lax.broadcasted_iota(jnp.int32, (12288, 8), 0)
        sign = jnp.where(row < N_POS // 8, 1.0, -1.0)
        z = grp * sign
        ls = jnp.minimum(z, 0.0) - jnp.log1p(jnp.exp(-jnp.abs(z)))
        o_ref[0, 0] = -jnp.sum(ls)

    return pl.pallas_call(
        body,
        out_shape=jax.ShapeDtypeStruct((1, 1), jnp.float32),
        out_specs=pl.BlockSpec(memory_space=pltpu.SMEM),
    )(s)


def kernel(pos_u, pos_v, neg_u, neg_v, u_table, v_table):
    u_idx = jnp.concatenate([pos_u, neg_u]).astype(jnp.int32).reshape(ROWS, 128)
    v_idx = jnp.concatenate([pos_v, neg_v]).astype(jnp.int32).reshape(ROWS, 128)
    scores16 = _sc_scores(u_idx, v_idx, u_table, v_table)
    loss = _tc_loss(scores16.reshape(12288, 128))
    return loss[0, 0]
